# Initial kernel scaffold; baseline (speedup 1.0000x reference)
import jax
import jax.numpy as jnp
from jax.experimental import pallas as pl


def kernel(node_feats, edge_feats, rdkitEF, edge_index, node_graph_ids, params):
    raise NotImplementedError("write your pallas kernel here")


# speedup vs baseline: 3.9877x; 3.9877x over previous
import functools

import jax
import jax.numpy as jnp
from jax import lax
from jax.experimental import pallas as pl
from jax.experimental.pallas import tpu as pltpu
from jax.experimental.pallas import tpu_sc as plsc

N = 10000
E = 160000
G = 512
DN = 128
DE = 16
D = 200
DP = 208
PA = 112
PB = 96
RW = 128
ROWS = E // RW
NC = 2
NS = 16
NW = NC * NS
RPW = -(-ROWS // NW)

_MESH = plsc.VectorSubcoreMesh(
    core_axis_name="c", subcore_axis_name="s", num_cores=NC, num_subcores=NS)
_SC_PARAMS = pltpu.CompilerParams(
    needs_layout_passes=False, use_tc_tiling_on_sc=False)

F32 = jnp.float32


def _lk(x):
    return jnp.where(x > 0, x, 0.01 * x)


def _elu(x):
    return jnp.where(x > 0, x, jnp.exp(jnp.minimum(x, 0.0)) - 1.0)


def _gru(x, h, w_ih, w_hh, b_ih, b_hh):
    gi = jnp.dot(x, w_ih, preferred_element_type=F32) + b_ih
    gh = jnp.dot(h, w_hh, preferred_element_type=F32) + b_hh
    r = jax.nn.sigmoid(gi[:, :D] + gh[:, :D])
    z = jax.nn.sigmoid(gi[:, D:2 * D] + gh[:, D:2 * D])
    n = jnp.tanh(gi[:, 2 * D:] + r * gh[:, 2 * D:])
    return (1.0 - z) * n + z * h


def _mm(a, b):
    return jnp.dot(a, b, preferred_element_type=F32)


def _wid():
    return lax.axis_index("s") * NC + lax.axis_index("c")


def _tc_node_pre_body(nf, wpn, bpn, w1t, b1, w2t, b2, hv_o, u_o, q_o):
    hv = _lk(_mm(nf[...], wpn[...]) + bpn[...])
    hv_o[...] = hv
    u_o[:, :D] = _mm(nf[...], w1t[...]) + b1[...]
    u_o[:, D:] = jnp.zeros((nf.shape[0], DP - D), F32)
    q_o[...] = _mm(hv, w2t[...]) + b2[...]


def _tc_node_pre(nf, wpn, bpn, w1t, b1, w2t, b2):
    blk = 2000
    grid = (N // blk,)
    full = lambda s: pl.BlockSpec(s, lambda i: tuple(0 for _ in s))
    return pl.pallas_call(
        _tc_node_pre_body,
        grid=grid,
        in_specs=[
            pl.BlockSpec((blk, DN), lambda i: (i, 0)),
            full((DN, D)), full((D,)), full((DN, D)), full((D,)),
            full((D, 1)), full((1,)),
        ],
        out_specs=[
            pl.BlockSpec((blk, D), lambda i: (i, 0)),
            pl.BlockSpec((blk, DP), lambda i: (i, 0)),
            pl.BlockSpec((blk, 1), lambda i: (i, 0)),
        ],
        out_shape=[
            jax.ShapeDtypeStruct((N, D), F32),
            jax.ShapeDtypeStruct((N, DP), F32),
            jax.ShapeDtypeStruct((N, 1), F32),
        ],
    )(nf, wpn, bpn, w1t, b1, w2t, b2)


def _tc_edge_body(u, ef, qd, w1b, w2b, weta, beta, wetb, betb,
                  pa_o, pb_o, e_o):
    he1 = _lk(u[:, :D] + _mm(ef[...], w1b[...]))
    pa_o[...] = _mm(he1, weta[...]) + beta[...]
    pb_o[:, :D - PA] = _mm(he1, wetb[...]) + betb[...]
    pb_o[:, D - PA:] = jnp.zeros((u.shape[0], PB - (D - PA)), F32)
    t = _mm(he1, w2b[...])
    e_o[...] = jnp.exp(_lk(qd[...] + t))


def _tc_edge(u_g, ef, qd, w1b, w2b, weta, beta, wetb, betb):
    blk = 2000
    grid = (E // blk,)
    full = lambda s: pl.BlockSpec(s, lambda i: tuple(0 for _ in s))
    return pl.pallas_call(
        _tc_edge_body,
        grid=grid,
        in_specs=[
            pl.BlockSpec((blk, DP), lambda i: (i, 0)),
            pl.BlockSpec((blk, DE), lambda i: (i, 0)),
            pl.BlockSpec((blk, 1), lambda i: (i, 0)),
            full((DE, D)), full((D, 1)),
            full((D, PA)), full((PA,)), full((D, D - PA)), full((D - PA,)),
        ],
        out_specs=[
            pl.BlockSpec((blk, PA), lambda i: (i, 0)),
            pl.BlockSpec((blk, PB), lambda i: (i, 0)),
            pl.BlockSpec((blk, 1), lambda i: (i, 0)),
        ],
        out_shape=[
            jax.ShapeDtypeStruct((E, PA), F32),
            jax.ShapeDtypeStruct((E, PB), F32),
            jax.ShapeDtypeStruct((E, 1), F32),
        ],
    )(u_g, ef, qd, w1b, w2b, weta, beta, wetb, betb)


def _tc_gru_body(a0, a1, b0, b1, hv, wih, whh, bih, bhh, h_o):
    ctx = _elu(jnp.concatenate(
        [a0[...] + a1[...], (b0[...] + b1[...])[:, :D - PA]], axis=1))
    h_o[...] = jax.nn.relu(
        _gru(ctx, hv[...], wih[...], whh[...], bih[...], bhh[...]))


def _tc_gru(ca, cb, hv, gru_p):
    blk = 2000
    grid = (N // blk,)
    full = lambda s: pl.BlockSpec(s, lambda i: tuple(0 for _ in s))
    return pl.pallas_call(
        _tc_gru_body,
        grid=grid,
        in_specs=[
            pl.BlockSpec((blk, PA), lambda i: (i, 0)),
            pl.BlockSpec((blk, PA), lambda i: (i, 0)),
            pl.BlockSpec((blk, PB), lambda i: (i, 0)),
            pl.BlockSpec((blk, PB), lambda i: (i, 0)),
            pl.BlockSpec((blk, D), lambda i: (i, 0)),
            full((D, 3 * D)), full((D, 3 * D)), full((3 * D,)), full((3 * D,)),
        ],
        out_specs=pl.BlockSpec((blk, D), lambda i: (i, 0)),
        out_shape=jax.ShapeDtypeStruct((N, D), F32),
    )(ca[0], ca[1], cb[0], cb[1], hv,
      gru_p['W_ih'], gru_p['W_hh'], gru_p['b_ih'], gru_p['b_hh'])


def _tc_l2node_body(h, wa, wb, bpe, wpna, bpna, wpnb, bpnb,
                    s1_o, s2_o, hvpa_o, hvpb_o):
    hh = h[...]
    s1_o[...] = _mm(hh, wa[...]) + bpe[...]
    s2_o[...] = _mm(hh, wb[...])
    hvpa_o[...] = _mm(hh, wpna[...]) + bpna[...]
    hvpb_o[:, :D - PA] = _mm(hh, wpnb[...]) + bpnb[...]
    hvpb_o[:, D - PA:] = jnp.zeros((h.shape[0], PB - (D - PA)), F32)


def _tc_l2node(h, wa, wb, bpe, wpna, bpna, wpnb, bpnb):
    blk = 2000
    grid = (N // blk,)
    full = lambda s: pl.BlockSpec(s, lambda i: tuple(0 for _ in s))
    return pl.pallas_call(
        _tc_l2node_body,
        grid=grid,
        in_specs=[
            pl.BlockSpec((blk, D), lambda i: (i, 0)),
            full((D, 1)), full((D, 1)), full((1,)),
            full((D, PA)), full((PA,)), full((D, D - PA)), full((D - PA,)),
        ],
        out_specs=[
            pl.BlockSpec((blk, 1), lambda i: (i, 0)),
            pl.BlockSpec((blk, 1), lambda i: (i, 0)),
            pl.BlockSpec((blk, PA), lambda i: (i, 0)),
            pl.BlockSpec((blk, PB), lambda i: (i, 0)),
        ],
        out_shape=[
            jax.ShapeDtypeStruct((N, 1), F32),
            jax.ShapeDtypeStruct((N, 1), F32),
            jax.ShapeDtypeStruct((N, PA), F32),
            jax.ShapeDtypeStruct((N, PB), F32),
        ],
    )(h, wa, wb, bpe, wpna, bpna, wpnb, bpnb)


_CH = 2000


def _af_chunk(gid, k):
    gk = gid[:, k * _CH:(k + 1) * _CH]
    giota = lax.broadcasted_iota(jnp.int32, (G, _CH), 0)
    amask = gk == giota
    return amask, amask.astype(F32)


def _tc_readout_body(h_r, gid_r, rd_r, *refs):
    (wz0a, wz0b, bz0, wp0, bp0, g0ih, g0hh, g0bi, g0bh,
     wz1a, wz1b, bz1, wp1, bp1, g1ih, g1hh, g1bi, g1bh,
     pw1, pb1, pg1, pbe1, pw2, pb2, pg2, pbe2, pw3, pb3, out_r, z_s) = refs
    gid = gid_r[...]
    nch = N // _CH

    g_feats = jnp.zeros((G, D), F32)
    for k in range(nch):
        _, af = _af_chunk(gid, k)
        g_feats = g_feats + _mm(af, h_r[k * _CH:(k + 1) * _CH, :])

    for (wza, wzb, bz, wp, bp, gih, ghh, gbi, gbh) in (
            (wz0a, wz0b, bz0, wp0, bp0, g0ih, g0hh, g0bi, g0bh),
            (wz1a, wz1b, bz1, wp1, bp1, g1ih, g1hh, g1bi, g1bh)):
        gr = jax.nn.relu(g_feats)
        zg = _mm(gr, wza[...])
        m = jnp.full((G, 1), -jnp.inf, F32)
        for k in range(nch):
            amask, af = _af_chunk(gid, k)
            hk = h_r[k * _CH:(k + 1) * _CH, :]
            zrow = _lk(
                lax.dot_general(zg, af, (((0,), (0,)), ((), ())),
                                preferred_element_type=F32)
                + lax.dot_general(wzb[...], hk, (((0,), (1,)), ((), ())),
                                  preferred_element_type=F32)
                + bz[...])
            z_s[:, k * _CH:(k + 1) * _CH] = zrow
            zb = jnp.broadcast_to(zrow, (G, _CH))
            m = jnp.maximum(
                m, jnp.max(jnp.where(amask, zb, -jnp.inf), axis=1,
                           keepdims=True))
        m = jnp.where(jnp.isfinite(m), m, 0.0)
        sacc = jnp.zeros((G, 1), F32)
        for k in range(nch):
            _, af = _af_chunk(gid, k)
            mrow = lax.dot_general(m, af, (((0,), (0,)), ((), ())),
                                   preferred_element_type=F32)
            erow = jnp.exp(z_s[:, k * _CH:(k + 1) * _CH] - mrow)
            z_s[:, k * _CH:(k + 1) * _CH] = erow
            sacc = sacc + lax.dot_general(af, erow, (((1,), (1,)), ((), ())),
                                          preferred_element_type=F32)
        gacc = jnp.zeros((G, D), F32)
        for k in range(nch):
            _, af = _af_chunk(gid, k)
            hk = h_r[k * _CH:(k + 1) * _CH, :]
            srow = lax.dot_general(sacc, af, (((0,), (0,)), ((), ())),
                                   preferred_element_type=F32)
            arow = z_s[:, k * _CH:(k + 1) * _CH] / (srow + 1e-16)
            hvp = _mm(hk, wp[...]) + bp[...]
            gacc = gacc + _mm(af * arow, hvp)
        g_repr = _elu(gacc)
        g_feats = _gru(jax.nn.relu(g_repr), g_feats,
                       gih[...], ghh[...], gbi[...], gbh[...])

    x = jnp.concatenate([g_feats, rd_r[...]], axis=1)
    x = jax.nn.relu(_mm(x, pw1[...]) + pb1[...])
    mu = jnp.mean(x, axis=0, keepdims=True)
    va = jnp.mean((x - mu) ** 2, axis=0, keepdims=True)
    x = (x - mu) / jnp.sqrt(va + 1e-5) * pg1[...] + pbe1[...]
    x = jax.nn.relu(_mm(x, pw2[...]) + pb2[...])
    mu = jnp.mean(x, axis=0, keepdims=True)
    va = jnp.mean((x - mu) ** 2, axis=0, keepdims=True)
    x = (x - mu) / jnp.sqrt(va + 1e-5) * pg2[...] + pbe2[...]
    out_r[...] = _mm(x, pw3[...]) + pb3[...]


def _tc_readout(h, gid_row, rdkit, p):
    args = [h, gid_row, rdkit]
    for t in range(2):
        args += [p['ro%d_W_z' % t][:D], p['ro%d_W_z' % t][D:],
                 p['ro%d_b_z' % t], p['ro%d_W_p' % t], p['ro%d_b_p' % t],
                 p['ro%d_gru' % t]['W_ih'], p['ro%d_gru' % t]['W_hh'],
                 p['ro%d_gru' % t]['b_ih'], p['ro%d_gru' % t]['b_hh']]
    args += [p['pd_W1'], p['pd_b1'], p['pd_g1'], p['pd_be1'],
             p['pd_W2'], p['pd_b2'], p['pd_g2'], p['pd_be2'],
             p['pd_W3'], p['pd_b3']]
    return pl.pallas_call(
        _tc_readout_body,
        out_shape=jax.ShapeDtypeStruct((G, 1), F32),
        scratch_shapes=[pltpu.VMEM((1, N), F32)],
    )(*args)


def _zero_rows(z2d, sh):
    def body(k, _):
        pltpu.sync_copy(z2d, sh.at[pl.ds(k * RW, RW)])
        return 0
    lax.fori_loop(0, N // RW, body, 0)
    pltpu.sync_copy(z2d.at[pl.ds(0, N % RW)], sh.at[pl.ds(N - N % RW, N % RW)])


def _zero_scal(z1d, sh):
    def body(k, _):
        pltpu.sync_copy(z1d, sh.at[pl.ds(k * 1024, 1024)])
        return 0
    lax.fori_loop(0, N // 1024, body, 0)
    pltpu.sync_copy(z1d.at[pl.ds(0, N % 1024)],
                    sh.at[pl.ds(N - N % 1024, N % 1024)])


def _per_worker_rows(body):
    w = _wid()

    def outer(i, _):
        row = w + NW * i

        @pl.when(row < ROWS)
        def _():
            body(row)
        return 0
    lax.fori_loop(0, RPW, outer, 0)


def _stage_recip(s_part, r, tmp):
    pltpu.sync_copy(s_part.at[0], r)
    pltpu.sync_copy(s_part.at[1], tmp)

    def body(j, _):
        sl = pl.ds(j * 16, 16)
        r[sl] = 1.0 / (r[sl] + tmp[sl] + 1e-16)
        return 0
    lax.fori_loop(0, N // 16, body, 0)


def _scale_rows(rows, av, w):
    def body(j, _):
        v = av[pl.ds(j, 16)]
        s = jnp.broadcast_to(v[0], (16,))
        for g in range(w // 16):
            sl = pl.ds(g * 16, 16)
            rows[j, sl] = rows[j, sl] * s
        return 0
    lax.fori_loop(0, RW, body, 0)


def _ctx_copy_out(ctx_sh, out, cid, sid):
    rpt = N // NS
    pltpu.sync_copy(ctx_sh.at[pl.ds(sid * rpt, rpt)],
                    out.at[cid, pl.ds(sid * rpt, rpt)])


@functools.partial(
    pl.kernel, mesh=_MESH, compiler_params=_SC_PARAMS,
    out_type=[jax.ShapeDtypeStruct((E, DP), F32),
              jax.ShapeDtypeStruct((ROWS, RW), F32)],
    scratch_types=[pltpu.VMEM((RW,), jnp.int32),
                   pltpu.VMEM((RW,), jnp.int32),
                   pltpu.VMEM((RW, DP), F32),
                   pltpu.VMEM((N,), F32),
                   pltpu.VMEM((RW,), F32),
                   pltpu.SemaphoreType.DMA],
)
def _sc_gather1(u_h, q_h, src2d, dst2d, u_out, qd_out,
                sidx, didx, rows, qtab, qd, sem):
    pltpu.sync_copy(q_h, qtab)

    def body(row):
        pltpu.sync_copy(src2d.at[row], sidx)
        pltpu.async_copy(u_h.at[sidx], rows, sem).wait()
        pltpu.sync_copy(rows, u_out.at[pl.ds(row * RW, RW)])
        pltpu.sync_copy(dst2d.at[row], didx)
        for g in range(RW // 16):
            qd[pl.ds(g * 16, 16)] = plsc.load_gather(
                qtab, [didx[pl.ds(g * 16, 16)]])
        pltpu.sync_copy(qd, qd_out.at[row])
    _per_worker_rows(body)


@functools.partial(
    pl.kernel, mesh=_MESH, compiler_params=_SC_PARAMS,
    out_type=jax.ShapeDtypeStruct((NC, N), F32),
    scratch_types=[pltpu.VMEM((RW,), jnp.int32),
                   pltpu.VMEM((RW,), F32),
                   pltpu.VMEM_SHARED((N,), F32)],
)
def _sc_scatter_e(e2d, dst2d, z1d, s_out, didx, ev, s_sh):
    cid = lax.axis_index("c")
    sid = lax.axis_index("s")

    @pl.when(sid == 0)
    def _():
        _zero_scal(z1d, s_sh)
    plsc.subcore_barrier()

    def body(row):
        pltpu.sync_copy(dst2d.at[row], didx)
        pltpu.sync_copy(e2d.at[row], ev)
        pltpu.sync_copy(ev, s_sh.at[didx], add=True)
    _per_worker_rows(body)
    plsc.subcore_barrier()

    @pl.when(sid == 0)
    def _():
        pltpu.sync_copy(s_sh, s_out.at[cid])


@functools.partial(
    pl.kernel, mesh=_MESH, compiler_params=_SC_PARAMS,
    out_type=jax.ShapeDtypeStruct((ROWS, RW), F32),
    scratch_types=[pltpu.VMEM((RW,), jnp.int32),
                   pltpu.VMEM((RW,), F32),
                   pltpu.VMEM((RW,), F32),
                   pltpu.VMEM((N,), F32),
                   pltpu.VMEM((N,), F32)],
)
def _sc_edge_coef(e2d, dst2d, s_part, a_out, didx, ev, avv, r, tmp):
    _stage_recip(s_part, r, tmp)

    def body(row):
        pltpu.sync_copy(dst2d.at[row], didx)
        pltpu.sync_copy(e2d.at[row], ev)
        for g in range(RW // 16):
            sl = pl.ds(g * 16, 16)
            rd = plsc.load_gather(r, [didx[sl]])
            avv[sl] = ev[sl] * rd
        pltpu.sync_copy(avv, a_out.at[row])
    _per_worker_rows(body)


def _make_scale_scatter(w):
    @functools.partial(
        pl.kernel, mesh=_MESH, compiler_params=_SC_PARAMS,
        out_type=jax.ShapeDtypeStruct((NC, N, w), F32),
        scratch_types=[pltpu.VMEM((RW,), jnp.int32),
                       pltpu.VMEM((RW + 16,), F32),
                       pltpu.VMEM((RW, w), F32),
                       pltpu.VMEM_SHARED((N, w), F32)],
    )
    def _k(p_h, a2d, dst2d, z2d, ctx_out, didx, av, rows, ctx_sh):
        cid = lax.axis_index("c")
        sid = lax.axis_index("s")

        @pl.when(sid == 0)
        def _():
            _zero_rows(z2d, ctx_sh)
        plsc.subcore_barrier()

        def body(row):
            pltpu.sync_copy(dst2d.at[row], didx)
            pltpu.sync_copy(a2d.at[row], av.at[pl.ds(0, RW)])
            pltpu.sync_copy(p_h.at[pl.ds(row * RW, RW)], rows)
            _scale_rows(rows, av, w)
            pltpu.sync_copy(rows, ctx_sh.at[didx], add=True)
        _per_worker_rows(body)
        plsc.subcore_barrier()
        _ctx_copy_out(ctx_sh, ctx_out, cid, sid)
    return _k


_SC_SCALE_SCATTER = {w: _make_scale_scatter(w) for w in (PA, PB)}


@functools.partial(
    pl.kernel, mesh=_MESH, compiler_params=_SC_PARAMS,
    out_type=[jax.ShapeDtypeStruct((ROWS, RW), F32),
              jax.ShapeDtypeStruct((NC, N), F32)],
    scratch_types=[pltpu.VMEM((RW,), jnp.int32),
                   pltpu.VMEM((RW,), jnp.int32),
                   pltpu.VMEM((RW,), F32),
                   pltpu.VMEM((N,), F32),
                   pltpu.VMEM((N,), F32),
                   pltpu.VMEM_SHARED((N,), F32)],
)
def _sc_edge_logits(s1_h, s2_h, src2d, dst2d, z1d, e_out, s_out,
                    didx, sidx, ev, t1, t2, s_sh):
    cid = lax.axis_index("c")
    sid = lax.axis_index("s")

    @pl.when(sid == 0)
    def _():
        _zero_scal(z1d, s_sh)
    pltpu.sync_copy(s1_h, t1)
    pltpu.sync_copy(s2_h, t2)
    plsc.subcore_barrier()

    def body(row):
        pltpu.sync_copy(dst2d.at[row], didx)
        pltpu.sync_copy(src2d.at[row], sidx)
        for g in range(RW // 16):
            sl = pl.ds(g * 16, 16)
            x = (plsc.load_gather(t1, [didx[sl]])
                 + plsc.load_gather(t2, [sidx[sl]]))
            ev[sl] = jnp.exp(jnp.where(x > 0, x, 0.01 * x))
        pltpu.sync_copy(ev, e_out.at[row])
        pltpu.sync_copy(ev, s_sh.at[didx], add=True)
    _per_worker_rows(body)
    plsc.subcore_barrier()

    @pl.when(sid == 0)
    def _():
        pltpu.sync_copy(s_sh, s_out.at[cid])


def _make_gather_scale_scatter(w):
    @functools.partial(
        pl.kernel, mesh=_MESH, compiler_params=_SC_PARAMS,
        out_type=jax.ShapeDtypeStruct((NC, N, w), F32),
        scratch_types=[pltpu.VMEM((RW,), jnp.int32),
                       pltpu.VMEM((RW,), jnp.int32),
                       pltpu.VMEM((RW + 16,), F32),
                       pltpu.VMEM((RW, w), F32),
                       pltpu.VMEM_SHARED((N, w), F32),
                       pltpu.SemaphoreType.DMA],
    )
    def _k(hvp_h, a2d, src2d, dst2d, z2d, ctx_out,
           didx, sidx, av, rows, ctx_sh, sem):
        cid = lax.axis_index("c")
        sid = lax.axis_index("s")

        @pl.when(sid == 0)
        def _():
            _zero_rows(z2d, ctx_sh)
        plsc.subcore_barrier()

        def body(row):
            pltpu.sync_copy(dst2d.at[row], didx)
            pltpu.sync_copy(src2d.at[row], sidx)
            pltpu.sync_copy(a2d.at[row], av.at[pl.ds(0, RW)])
            pltpu.async_copy(hvp_h.at[sidx], rows, sem).wait()
            _scale_rows(rows, av, w)
            pltpu.sync_copy(rows, ctx_sh.at[didx], add=True)
        _per_worker_rows(body)
        plsc.subcore_barrier()
        _ctx_copy_out(ctx_sh, ctx_out, cid, sid)
    return _k


_SC_GATHER_SCALE_SCATTER = {w: _make_gather_scale_scatter(w) for w in (PA, PB)}


def kernel(node_feats, edge_feats, rdkitEF, edge_index, node_graph_ids, params):
    p = params
    src = edge_index[0].astype(jnp.int32)
    dst = edge_index[1].astype(jnp.int32)
    src2d = src.reshape(ROWS, RW)
    dst2d = dst.reshape(ROWS, RW)
    z1d = jnp.zeros((1024,), F32)
    za = jnp.zeros((RW, PA), F32)
    zb = jnp.zeros((RW, PB), F32)
    gid_row = node_graph_ids.astype(jnp.int32).reshape(1, N)

    hv, u, q = _tc_node_pre(
        node_feats, p['gc_W_pn'], p['gc_b_pn'],
        p['gc_W_pe1'][:DN], p['gc_b_pe1'],
        p['gc_W_pe2'][:D], p['gc_b_pe2'])
    u_g, qd2d = _sc_gather1(u, q.reshape(N), src2d, dst2d)
    pa, pb, e = _tc_edge(u_g, edge_feats, qd2d.reshape(E, 1),
                         p['gc_W_pe1'][DN:], p['gc_W_pe2'][D:],
                         p['gc_W_et'][:, :PA], p['gc_b_et'][:PA],
                         p['gc_W_et'][:, PA:], p['gc_b_et'][PA:])
    e2d = e.reshape(ROWS, RW)
    s_part = _sc_scatter_e(e2d, dst2d, z1d)
    a2d = _sc_edge_coef(e2d, dst2d, s_part)
    ctxa = _SC_SCALE_SCATTER[PA](pa, a2d, dst2d, za)
    ctxb = _SC_SCALE_SCATTER[PB](pb, a2d, dst2d, zb)
    h = _tc_gru(ctxa, ctxb, hv, p['gc_gru'])

    s1, s2, hvpa, hvpb = _tc_l2node(
        h, p['gl_W_pe'][:D], p['gl_W_pe'][D:], p['gl_b_pe'],
        p['gl_W_pn'][:, :PA], p['gl_b_pn'][:PA],
        p['gl_W_pn'][:, PA:], p['gl_b_pn'][PA:])
    e2_2d, s2_part = _sc_edge_logits(s1.reshape(N), s2.reshape(N),
                                     src2d, dst2d, z1d)
    a2_2d = _sc_edge_coef(e2_2d, dst2d, s2_part)
    ctx2a = _SC_GATHER_SCALE_SCATTER[PA](hvpa, a2_2d, src2d, dst2d, za)
    ctx2b = _SC_GATHER_SCALE_SCATTER[PB](hvpb, a2_2d, src2d, dst2d, zb)
    h2 = _tc_gru(ctx2a, ctx2b, h, p['gl_gru'])

    return _tc_readout(h2, gid_row, rdkitEF, p)


# speedup vs baseline: 4.4140x; 1.1069x over previous
import functools

import jax
import jax.numpy as jnp
from jax import lax
from jax.experimental import pallas as pl
from jax.experimental.pallas import tpu as pltpu
from jax.experimental.pallas import tpu_sc as plsc

N = 10000
E = 160000
G = 512
DN = 128
DE = 16
D = 200
DP = 208
PA = 112
PB = 96
RW = 128
ROWS = E // RW
NC = 2
NS = 16
NW = NC * NS
RPW = -(-ROWS // NW)

_MESH = plsc.VectorSubcoreMesh(
    core_axis_name="c", subcore_axis_name="s", num_cores=NC, num_subcores=NS)
_SC_PARAMS = pltpu.CompilerParams(
    needs_layout_passes=False, use_tc_tiling_on_sc=False)

F32 = jnp.float32


def _lk(x):
    return jnp.where(x > 0, x, 0.01 * x)


def _elu(x):
    return jnp.where(x > 0, x, jnp.exp(jnp.minimum(x, 0.0)) - 1.0)


def _gru(x, h, w_ih, w_hh, b_ih, b_hh):
    gi = jnp.dot(x, w_ih, preferred_element_type=F32) + b_ih
    gh = jnp.dot(h, w_hh, preferred_element_type=F32) + b_hh
    r = jax.nn.sigmoid(gi[:, :D] + gh[:, :D])
    z = jax.nn.sigmoid(gi[:, D:2 * D] + gh[:, D:2 * D])
    n = jnp.tanh(gi[:, 2 * D:] + r * gh[:, 2 * D:])
    return (1.0 - z) * n + z * h


def _mm(a, b):
    return jnp.dot(a, b, preferred_element_type=F32)


def _wid():
    return lax.axis_index("s") * NC + lax.axis_index("c")


def _tc_node_pre_body(nf, wpn, bpn, w1t, b1, w2t, b2, hv_o, u_o, q_o):
    hv = _lk(_mm(nf[...], wpn[...]) + bpn[...])
    hv_o[...] = hv
    u_o[:, :D] = _mm(nf[...], w1t[...]) + b1[...]
    u_o[:, D:] = jnp.zeros((nf.shape[0], DP - D), F32)
    q_o[...] = _mm(hv, w2t[...]) + b2[...]


def _tc_node_pre(nf, wpn, bpn, w1t, b1, w2t, b2):
    blk = 2000
    grid = (N // blk,)
    full = lambda s: pl.BlockSpec(s, lambda i: tuple(0 for _ in s))
    return pl.pallas_call(
        _tc_node_pre_body,
        grid=grid,
        in_specs=[
            pl.BlockSpec((blk, DN), lambda i: (i, 0)),
            full((DN, D)), full((D,)), full((DN, D)), full((D,)),
            full((D, 1)), full((1,)),
        ],
        out_specs=[
            pl.BlockSpec((blk, D), lambda i: (i, 0)),
            pl.BlockSpec((blk, DP), lambda i: (i, 0)),
            pl.BlockSpec((blk, 1), lambda i: (i, 0)),
        ],
        out_shape=[
            jax.ShapeDtypeStruct((N, D), F32),
            jax.ShapeDtypeStruct((N, DP), F32),
            jax.ShapeDtypeStruct((N, 1), F32),
        ],
    )(nf, wpn, bpn, w1t, b1, w2t, b2)


def _tc_edge_body(u, ef, qd, w1b, w2b, weta, beta, wetb, betb,
                  pa_o, pb_o, e_o):
    he1 = _lk(u[:, :D] + _mm(ef[...], w1b[...]))
    t = _mm(he1, w2b[...])
    e = jnp.exp(_lk(qd[...] + t))
    e_o[...] = e
    pa_o[...] = (_mm(he1, weta[...]) + beta[...]) * e
    pb_o[:, :D - PA] = (_mm(he1, wetb[...]) + betb[...]) * e
    pb_o[:, D - PA:] = jnp.zeros((u.shape[0], PB - (D - PA)), F32)


def _tc_edge(u_g, ef, qd, w1b, w2b, weta, beta, wetb, betb):
    blk = 2000
    grid = (E // blk,)
    full = lambda s: pl.BlockSpec(s, lambda i: tuple(0 for _ in s))
    return pl.pallas_call(
        _tc_edge_body,
        grid=grid,
        in_specs=[
            pl.BlockSpec((blk, DP), lambda i: (i, 0)),
            pl.BlockSpec((blk, DE), lambda i: (i, 0)),
            pl.BlockSpec((blk, 1), lambda i: (i, 0)),
            full((DE, D)), full((D, 1)),
            full((D, PA)), full((PA,)), full((D, D - PA)), full((D - PA,)),
        ],
        out_specs=[
            pl.BlockSpec((blk, PA), lambda i: (i, 0)),
            pl.BlockSpec((blk, PB), lambda i: (i, 0)),
            pl.BlockSpec((blk, 1), lambda i: (i, 0)),
        ],
        out_shape=[
            jax.ShapeDtypeStruct((E, PA), F32),
            jax.ShapeDtypeStruct((E, PB), F32),
            jax.ShapeDtypeStruct((E, 1), F32),
        ],
    )(u_g, ef, qd, w1b, w2b, weta, beta, wetb, betb)


def _tc_gru_body(a0, a1, b0, b1, sp0, sp1, hv, wih, whh, bih, bhh, h_o):
    den = sp0[...] + sp1[...] + 1e-16
    ctx = _elu(jnp.concatenate(
        [a0[...] + a1[...], (b0[...] + b1[...])[:, :D - PA]], axis=1) / den)
    h_o[...] = jax.nn.relu(
        _gru(ctx, hv[...], wih[...], whh[...], bih[...], bhh[...]))


def _tc_gru(ca, cb, s_part, hv, gru_p):
    blk = 2000
    grid = (N // blk,)
    full = lambda s: pl.BlockSpec(s, lambda i: tuple(0 for _ in s))
    return pl.pallas_call(
        _tc_gru_body,
        grid=grid,
        in_specs=[
            pl.BlockSpec((blk, PA), lambda i: (i, 0)),
            pl.BlockSpec((blk, PA), lambda i: (i, 0)),
            pl.BlockSpec((blk, PB), lambda i: (i, 0)),
            pl.BlockSpec((blk, PB), lambda i: (i, 0)),
            pl.BlockSpec((blk, 1), lambda i: (i, 0)),
            pl.BlockSpec((blk, 1), lambda i: (i, 0)),
            pl.BlockSpec((blk, D), lambda i: (i, 0)),
            full((D, 3 * D)), full((D, 3 * D)), full((3 * D,)), full((3 * D,)),
        ],
        out_specs=pl.BlockSpec((blk, D), lambda i: (i, 0)),
        out_shape=jax.ShapeDtypeStruct((N, D), F32),
    )(ca[0], ca[1], cb[0], cb[1],
      s_part[0].reshape(N, 1), s_part[1].reshape(N, 1), hv,
      gru_p['W_ih'], gru_p['W_hh'], gru_p['b_ih'], gru_p['b_hh'])


def _tc_l2node_body(h, wa, wb, bpe, wpna, bpna, wpnb, bpnb,
                    s1_o, s2_o, hvpa_o, hvpb_o):
    hh = h[...]
    s1_o[...] = _mm(hh, wa[...]) + bpe[...]
    s2_o[...] = _mm(hh, wb[...])
    hvpa_o[...] = _mm(hh, wpna[...]) + bpna[...]
    hvpb_o[:, :D - PA] = _mm(hh, wpnb[...]) + bpnb[...]
    hvpb_o[:, D - PA:] = jnp.zeros((h.shape[0], PB - (D - PA)), F32)


def _tc_l2node(h, wa, wb, bpe, wpna, bpna, wpnb, bpnb):
    blk = 2000
    grid = (N // blk,)
    full = lambda s: pl.BlockSpec(s, lambda i: tuple(0 for _ in s))
    return pl.pallas_call(
        _tc_l2node_body,
        grid=grid,
        in_specs=[
            pl.BlockSpec((blk, D), lambda i: (i, 0)),
            full((D, 1)), full((D, 1)), full((1,)),
            full((D, PA)), full((PA,)), full((D, D - PA)), full((D - PA,)),
        ],
        out_specs=[
            pl.BlockSpec((blk, 1), lambda i: (i, 0)),
            pl.BlockSpec((blk, 1), lambda i: (i, 0)),
            pl.BlockSpec((blk, PA), lambda i: (i, 0)),
            pl.BlockSpec((blk, PB), lambda i: (i, 0)),
        ],
        out_shape=[
            jax.ShapeDtypeStruct((N, 1), F32),
            jax.ShapeDtypeStruct((N, 1), F32),
            jax.ShapeDtypeStruct((N, PA), F32),
            jax.ShapeDtypeStruct((N, PB), F32),
        ],
    )(h, wa, wb, bpe, wpna, bpna, wpnb, bpnb)


_CH = 2000


def _af_chunk(gid, k):
    gk = gid[:, k * _CH:(k + 1) * _CH]
    giota = lax.broadcasted_iota(jnp.int32, (G, _CH), 0)
    amask = gk == giota
    return amask, amask.astype(F32)


def _tc_readout_body(h_r, gid_r, rd_r, *refs):
    (wz0a, wz0b, bz0, wp0, bp0, g0ih, g0hh, g0bi, g0bh,
     wz1a, wz1b, bz1, wp1, bp1, g1ih, g1hh, g1bi, g1bh,
     pw1, pb1, pg1, pbe1, pw2, pb2, pg2, pbe2, pw3, pb3, out_r, z_s) = refs
    gid = gid_r[...]
    nch = N // _CH

    g_feats = jnp.zeros((G, D), F32)
    for k in range(nch):
        _, af = _af_chunk(gid, k)
        g_feats = g_feats + _mm(af, h_r[k * _CH:(k + 1) * _CH, :])

    for (wza, wzb, bz, wp, bp, gih, ghh, gbi, gbh) in (
            (wz0a, wz0b, bz0, wp0, bp0, g0ih, g0hh, g0bi, g0bh),
            (wz1a, wz1b, bz1, wp1, bp1, g1ih, g1hh, g1bi, g1bh)):
        gr = jax.nn.relu(g_feats)
        zg = _mm(gr, wza[...])
        m = jnp.full((G, 1), -jnp.inf, F32)
        for k in range(nch):
            amask, af = _af_chunk(gid, k)
            hk = h_r[k * _CH:(k + 1) * _CH, :]
            zrow = _lk(
                lax.dot_general(zg, af, (((0,), (0,)), ((), ())),
                                preferred_element_type=F32)
                + lax.dot_general(wzb[...], hk, (((0,), (1,)), ((), ())),
                                  preferred_element_type=F32)
                + bz[...])
            z_s[:, k * _CH:(k + 1) * _CH] = zrow
            zb = jnp.broadcast_to(zrow, (G, _CH))
            m = jnp.maximum(
                m, jnp.max(jnp.where(amask, zb, -jnp.inf), axis=1,
                           keepdims=True))
        m = jnp.where(jnp.isfinite(m), m, 0.0)
        sacc = jnp.zeros((G, 1), F32)
        for k in range(nch):
            _, af = _af_chunk(gid, k)
            mrow = lax.dot_general(m, af, (((0,), (0,)), ((), ())),
                                   preferred_element_type=F32)
            erow = jnp.exp(z_s[:, k * _CH:(k + 1) * _CH] - mrow)
            z_s[:, k * _CH:(k + 1) * _CH] = erow
            sacc = sacc + lax.dot_general(af, erow, (((1,), (1,)), ((), ())),
                                          preferred_element_type=F32)
        gacc = jnp.zeros((G, D), F32)
        for k in range(nch):
            _, af = _af_chunk(gid, k)
            hk = h_r[k * _CH:(k + 1) * _CH, :]
            srow = lax.dot_general(sacc, af, (((0,), (0,)), ((), ())),
                                   preferred_element_type=F32)
            arow = z_s[:, k * _CH:(k + 1) * _CH] / (srow + 1e-16)
            hvp = _mm(hk, wp[...]) + bp[...]
            gacc = gacc + _mm(af * arow, hvp)
        g_repr = _elu(gacc)
        g_feats = _gru(jax.nn.relu(g_repr), g_feats,
                       gih[...], ghh[...], gbi[...], gbh[...])

    x = jnp.concatenate([g_feats, rd_r[...]], axis=1)
    x = jax.nn.relu(_mm(x, pw1[...]) + pb1[...])
    mu = jnp.mean(x, axis=0, keepdims=True)
    va = jnp.mean((x - mu) ** 2, axis=0, keepdims=True)
    x = (x - mu) / jnp.sqrt(va + 1e-5) * pg1[...] + pbe1[...]
    x = jax.nn.relu(_mm(x, pw2[...]) + pb2[...])
    mu = jnp.mean(x, axis=0, keepdims=True)
    va = jnp.mean((x - mu) ** 2, axis=0, keepdims=True)
    x = (x - mu) / jnp.sqrt(va + 1e-5) * pg2[...] + pbe2[...]
    out_r[...] = _mm(x, pw3[...]) + pb3[...]


def _tc_readout(h, gid_row, rdkit, p):
    args = [h, gid_row, rdkit]
    for t in range(2):
        args += [p['ro%d_W_z' % t][:D], p['ro%d_W_z' % t][D:],
                 p['ro%d_b_z' % t], p['ro%d_W_p' % t], p['ro%d_b_p' % t],
                 p['ro%d_gru' % t]['W_ih'], p['ro%d_gru' % t]['W_hh'],
                 p['ro%d_gru' % t]['b_ih'], p['ro%d_gru' % t]['b_hh']]
    args += [p['pd_W1'], p['pd_b1'], p['pd_g1'], p['pd_be1'],
             p['pd_W2'], p['pd_b2'], p['pd_g2'], p['pd_be2'],
             p['pd_W3'], p['pd_b3']]
    return pl.pallas_call(
        _tc_readout_body,
        out_shape=jax.ShapeDtypeStruct((G, 1), F32),
        scratch_shapes=[pltpu.VMEM((1, N), F32)],
    )(*args)


def _zero_rows(z2d, sh):
    def body(k, _):
        pltpu.sync_copy(z2d, sh.at[pl.ds(k * RW, RW)])
        return 0
    lax.fori_loop(0, N // RW, body, 0)
    pltpu.sync_copy(z2d.at[pl.ds(0, N % RW)], sh.at[pl.ds(N - N % RW, N % RW)])


def _zero_scal(z1d, sh):
    def body(k, _):
        pltpu.sync_copy(z1d, sh.at[pl.ds(k * 1024, 1024)])
        return 0
    lax.fori_loop(0, N // 1024, body, 0)
    pltpu.sync_copy(z1d.at[pl.ds(0, N % 1024)],
                    sh.at[pl.ds(N - N % 1024, N % 1024)])


def _per_worker_rows(body):
    w = _wid()

    def outer(i, _):
        row = w + NW * i

        @pl.when(row < ROWS)
        def _():
            body(row)
        return 0
    lax.fori_loop(0, RPW, outer, 0)


def _scale_rows(rows, av, w):
    def body(j, _):
        v = av[pl.ds(j, 16)]
        s = jnp.broadcast_to(v[0], (16,))
        for g in range(w // 16):
            sl = pl.ds(g * 16, 16)
            rows[j, sl] = rows[j, sl] * s
        return 0
    lax.fori_loop(0, RW, body, 0)


def _ctx_copy_out(ctx_sh, out, cid, sid):
    rpt = N // NS
    pltpu.sync_copy(ctx_sh.at[pl.ds(sid * rpt, rpt)],
                    out.at[cid, pl.ds(sid * rpt, rpt)])


@functools.partial(
    pl.kernel, mesh=_MESH, compiler_params=_SC_PARAMS,
    out_type=[jax.ShapeDtypeStruct((E, DP), F32),
              jax.ShapeDtypeStruct((ROWS, RW), F32)],
    scratch_types=[pltpu.VMEM((RW,), jnp.int32),
                   pltpu.VMEM((RW,), jnp.int32),
                   pltpu.VMEM((RW, DP), F32),
                   pltpu.VMEM((N,), F32),
                   pltpu.VMEM((RW,), F32),
                   pltpu.SemaphoreType.DMA],
)
def _sc_gather1(u_h, q_h, src2d, dst2d, u_out, qd_out,
                sidx, didx, rows, qtab, qd, sem):
    pltpu.sync_copy(q_h, qtab)

    def body(row):
        pltpu.sync_copy(src2d.at[row], sidx)
        pltpu.async_copy(u_h.at[sidx], rows, sem).wait()
        pltpu.sync_copy(rows, u_out.at[pl.ds(row * RW, RW)])
        pltpu.sync_copy(dst2d.at[row], didx)
        for g in range(RW // 16):
            qd[pl.ds(g * 16, 16)] = plsc.load_gather(
                qtab, [didx[pl.ds(g * 16, 16)]])
        pltpu.sync_copy(qd, qd_out.at[row])
    _per_worker_rows(body)


@functools.partial(
    pl.kernel, mesh=_MESH, compiler_params=_SC_PARAMS,
    out_type=[jax.ShapeDtypeStruct((NC, N, PA), F32),
              jax.ShapeDtypeStruct((NC, N), F32)],
    scratch_types=[pltpu.VMEM((RW,), jnp.int32),
                   pltpu.VMEM((RW,), F32),
                   pltpu.VMEM((RW, PA), F32),
                   pltpu.VMEM_SHARED((N, PA), F32),
                   pltpu.VMEM_SHARED((N,), F32)],
)
def _sc_scatter_rows_s(p_h, e2d, dst2d, z2d, z1d, ctx_out, s_out,
                       didx, ev, rows, ctx_sh, s_sh):
    cid = lax.axis_index("c")
    sid = lax.axis_index("s")

    @pl.when(sid == 0)
    def _():
        _zero_rows(z2d, ctx_sh)
        _zero_scal(z1d, s_sh)
    plsc.subcore_barrier()

    def body(row):
        pltpu.sync_copy(dst2d.at[row], didx)
        pltpu.sync_copy(p_h.at[pl.ds(row * RW, RW)], rows)
        pltpu.sync_copy(rows, ctx_sh.at[didx], add=True)
        pltpu.sync_copy(e2d.at[row], ev)
        pltpu.sync_copy(ev, s_sh.at[didx], add=True)
    _per_worker_rows(body)
    plsc.subcore_barrier()

    @pl.when(sid == 0)
    def _():
        pltpu.sync_copy(s_sh, s_out.at[cid])
    _ctx_copy_out(ctx_sh, ctx_out, cid, sid)


@functools.partial(
    pl.kernel, mesh=_MESH, compiler_params=_SC_PARAMS,
    out_type=jax.ShapeDtypeStruct((NC, N, PB), F32),
    scratch_types=[pltpu.VMEM((RW,), jnp.int32),
                   pltpu.VMEM((RW, PB), F32),
                   pltpu.VMEM_SHARED((N, PB), F32)],
)
def _sc_scatter_rows(p_h, dst2d, z2d, ctx_out, didx, rows, ctx_sh):
    cid = lax.axis_index("c")
    sid = lax.axis_index("s")

    @pl.when(sid == 0)
    def _():
        _zero_rows(z2d, ctx_sh)
    plsc.subcore_barrier()

    def body(row):
        pltpu.sync_copy(dst2d.at[row], didx)
        pltpu.sync_copy(p_h.at[pl.ds(row * RW, RW)], rows)
        pltpu.sync_copy(rows, ctx_sh.at[didx], add=True)
    _per_worker_rows(body)
    plsc.subcore_barrier()
    _ctx_copy_out(ctx_sh, ctx_out, cid, sid)


@functools.partial(
    pl.kernel, mesh=_MESH, compiler_params=_SC_PARAMS,
    out_type=[jax.ShapeDtypeStruct((ROWS, RW), F32),
              jax.ShapeDtypeStruct((NC, N), F32)],
    scratch_types=[pltpu.VMEM((RW,), jnp.int32),
                   pltpu.VMEM((RW,), jnp.int32),
                   pltpu.VMEM((RW,), F32),
                   pltpu.VMEM((N,), F32),
                   pltpu.VMEM((N,), F32),
                   pltpu.VMEM_SHARED((N,), F32)],
)
def _sc_edge_logits(s1_h, s2_h, src2d, dst2d, z1d, e_out, s_out,
                    didx, sidx, ev, t1, t2, s_sh):
    cid = lax.axis_index("c")
    sid = lax.axis_index("s")

    @pl.when(sid == 0)
    def _():
        _zero_scal(z1d, s_sh)
    pltpu.sync_copy(s1_h, t1)
    pltpu.sync_copy(s2_h, t2)
    plsc.subcore_barrier()

    def body(row):
        pltpu.sync_copy(dst2d.at[row], didx)
        pltpu.sync_copy(src2d.at[row], sidx)
        for g in range(RW // 16):
            sl = pl.ds(g * 16, 16)
            x = (plsc.load_gather(t1, [didx[sl]])
                 + plsc.load_gather(t2, [sidx[sl]]))
            ev[sl] = jnp.exp(jnp.where(x > 0, x, 0.01 * x))
        pltpu.sync_copy(ev, e_out.at[row])
        pltpu.sync_copy(ev, s_sh.at[didx], add=True)
    _per_worker_rows(body)
    plsc.subcore_barrier()

    @pl.when(sid == 0)
    def _():
        pltpu.sync_copy(s_sh, s_out.at[cid])


def _make_gather_scale_scatter(w):
    @functools.partial(
        pl.kernel, mesh=_MESH, compiler_params=_SC_PARAMS,
        out_type=jax.ShapeDtypeStruct((NC, N, w), F32),
        scratch_types=[pltpu.VMEM((RW,), jnp.int32),
                       pltpu.VMEM((RW,), jnp.int32),
                       pltpu.VMEM((RW + 16,), F32),
                       pltpu.VMEM((RW, w), F32),
                       pltpu.VMEM_SHARED((N, w), F32),
                       pltpu.SemaphoreType.DMA],
    )
    def _k(hvp_h, a2d, src2d, dst2d, z2d, ctx_out,
           didx, sidx, av, rows, ctx_sh, sem):
        cid = lax.axis_index("c")
        sid = lax.axis_index("s")

        @pl.when(sid == 0)
        def _():
            _zero_rows(z2d, ctx_sh)
        plsc.subcore_barrier()

        def body(row):
            pltpu.sync_copy(dst2d.at[row], didx)
            pltpu.sync_copy(src2d.at[row], sidx)
            pltpu.sync_copy(a2d.at[row], av.at[pl.ds(0, RW)])
            pltpu.async_copy(hvp_h.at[sidx], rows, sem).wait()
            _scale_rows(rows, av, w)
            pltpu.sync_copy(rows, ctx_sh.at[didx], add=True)
        _per_worker_rows(body)
        plsc.subcore_barrier()
        _ctx_copy_out(ctx_sh, ctx_out, cid, sid)
    return _k


_SC_GATHER_SCALE_SCATTER = {w: _make_gather_scale_scatter(w) for w in (PA, PB)}


def kernel(node_feats, edge_feats, rdkitEF, edge_index, node_graph_ids, params):
    p = params
    src = edge_index[0].astype(jnp.int32)
    dst = edge_index[1].astype(jnp.int32)
    src2d = src.reshape(ROWS, RW)
    dst2d = dst.reshape(ROWS, RW)
    z1d = jnp.zeros((1024,), F32)
    za = jnp.zeros((RW, PA), F32)
    zb = jnp.zeros((RW, PB), F32)
    gid_row = node_graph_ids.astype(jnp.int32).reshape(1, N)

    hv, u, q = _tc_node_pre(
        node_feats, p['gc_W_pn'], p['gc_b_pn'],
        p['gc_W_pe1'][:DN], p['gc_b_pe1'],
        p['gc_W_pe2'][:D], p['gc_b_pe2'])
    u_g, qd2d = _sc_gather1(u, q.reshape(N), src2d, dst2d)
    pa, pb, e = _tc_edge(u_g, edge_feats, qd2d.reshape(E, 1),
                         p['gc_W_pe1'][DN:], p['gc_W_pe2'][D:],
                         p['gc_W_et'][:, :PA], p['gc_b_et'][:PA],
                         p['gc_W_et'][:, PA:], p['gc_b_et'][PA:])
    e2d = e.reshape(ROWS, RW)
    ctxa, s_part = _sc_scatter_rows_s(pa, e2d, dst2d, za, z1d)
    ctxb = _sc_scatter_rows(pb, dst2d, zb)
    h = _tc_gru(ctxa, ctxb, s_part, hv, p['gc_gru'])

    s1, s2, hvpa, hvpb = _tc_l2node(
        h, p['gl_W_pe'][:D], p['gl_W_pe'][D:], p['gl_b_pe'],
        p['gl_W_pn'][:, :PA], p['gl_b_pn'][:PA],
        p['gl_W_pn'][:, PA:], p['gl_b_pn'][PA:])
    e2_2d, s2_part = _sc_edge_logits(s1.reshape(N), s2.reshape(N),
                                     src2d, dst2d, z1d)
    ctx2a = _SC_GATHER_SCALE_SCATTER[PA](hvpa, e2_2d, src2d, dst2d, za)
    ctx2b = _SC_GATHER_SCALE_SCATTER[PB](hvpb, e2_2d, src2d, dst2d, zb)
    h2 = _tc_gru(ctx2a, ctx2b, s2_part, h, p['gl_gru'])

    return _tc_readout(h2, gid_row, rdkitEF, p)


# speedup vs baseline: 4.6477x; 1.0530x over previous
import functools

import jax
import jax.numpy as jnp
from jax import lax
from jax.experimental import pallas as pl
from jax.experimental.pallas import tpu as pltpu
from jax.experimental.pallas import tpu_sc as plsc

N = 10000
E = 160000
G = 512
DN = 128
DE = 16
D = 200
DP = 208
PA = 112
PB = 96
RW = 128
ROWS = E // RW
NC = 2
NS = 16
NW = NC * NS
RPW = -(-ROWS // NW)

_MESH = plsc.VectorSubcoreMesh(
    core_axis_name="c", subcore_axis_name="s", num_cores=NC, num_subcores=NS)
_SC_PARAMS = pltpu.CompilerParams(
    needs_layout_passes=False, use_tc_tiling_on_sc=False)

F32 = jnp.float32


def _lk(x):
    return jnp.where(x > 0, x, 0.01 * x)


def _elu(x):
    return jnp.where(x > 0, x, jnp.exp(jnp.minimum(x, 0.0)) - 1.0)


def _gru(x, h, w_ih, w_hh, b_ih, b_hh):
    gi = jnp.dot(x, w_ih, preferred_element_type=F32) + b_ih
    gh = jnp.dot(h, w_hh, preferred_element_type=F32) + b_hh
    r = jax.nn.sigmoid(gi[:, :D] + gh[:, :D])
    z = jax.nn.sigmoid(gi[:, D:2 * D] + gh[:, D:2 * D])
    n = jnp.tanh(gi[:, 2 * D:] + r * gh[:, 2 * D:])
    return (1.0 - z) * n + z * h


def _mm(a, b):
    return jnp.dot(a, b, preferred_element_type=F32)


def _wid():
    return lax.axis_index("s") * NC + lax.axis_index("c")


def _tc_node_pre_body(nf, wpn, bpn, w1t, b1, w2t, b2, hv_o, u_o, q_o):
    hv = _lk(_mm(nf[...], wpn[...]) + bpn[...])
    hv_o[...] = hv
    u_o[:, :D] = _mm(nf[...], w1t[...]) + b1[...]
    u_o[:, D:] = jnp.zeros((nf.shape[0], DP - D), F32)
    q_o[...] = _mm(hv, w2t[...]) + b2[...]


def _tc_node_pre(nf, wpn, bpn, w1t, b1, w2t, b2):
    blk = 2000
    grid = (N // blk,)
    full = lambda s: pl.BlockSpec(s, lambda i: tuple(0 for _ in s))
    return pl.pallas_call(
        _tc_node_pre_body,
        grid=grid,
        in_specs=[
            pl.BlockSpec((blk, DN), lambda i: (i, 0)),
            full((DN, D)), full((D,)), full((DN, D)), full((D,)),
            full((D, 1)), full((1,)),
        ],
        out_specs=[
            pl.BlockSpec((blk, D), lambda i: (i, 0)),
            pl.BlockSpec((blk, DP), lambda i: (i, 0)),
            pl.BlockSpec((blk, 1), lambda i: (i, 0)),
        ],
        out_shape=[
            jax.ShapeDtypeStruct((N, D), F32),
            jax.ShapeDtypeStruct((N, DP), F32),
            jax.ShapeDtypeStruct((N, 1), F32),
        ],
    )(nf, wpn, bpn, w1t, b1, w2t, b2)


def _tc_edge_body(u, ef, qd, w1b, w2b, pa_o, pb_o):
    he1 = _lk(u[:, :D] + _mm(ef[...], w1b[...]))
    t = _mm(he1, w2b[...])
    e = jnp.exp(_lk(qd[...] + t))
    pa_o[...] = he1[:, :PA] * e
    pb_o[...] = jnp.concatenate(
        [he1[:, PA:] * e, e,
         jnp.zeros((u.shape[0], PB - (D - PA) - 1), F32)], axis=1)


def _tc_edge(u_g, ef, qd, w1b, w2b):
    blk = 2000
    grid = (E // blk,)
    full = lambda s: pl.BlockSpec(s, lambda i: tuple(0 for _ in s))
    return pl.pallas_call(
        _tc_edge_body,
        grid=grid,
        in_specs=[
            pl.BlockSpec((blk, DP), lambda i: (i, 0)),
            pl.BlockSpec((blk, DE), lambda i: (i, 0)),
            pl.BlockSpec((blk, 1), lambda i: (i, 0)),
            full((DE, D)), full((D, 1)),
        ],
        out_specs=[
            pl.BlockSpec((blk, PA), lambda i: (i, 0)),
            pl.BlockSpec((blk, PB), lambda i: (i, 0)),
        ],
        out_shape=[
            jax.ShapeDtypeStruct((E, PA), F32),
            jax.ShapeDtypeStruct((E, PB), F32),
        ],
    )(u_g, ef, qd, w1b, w2b)


def _ctx_from_panels(a0, a1, b0, b1, wlin, blin):
    bsum = b0[...] + b1[...]
    s = bsum[:, D - PA:D - PA + 1]
    den = s + 1e-16
    hsum = jnp.concatenate(
        [a0[...] + a1[...], bsum[:, :D - PA]], axis=1) / den
    return _elu(_mm(hsum, wlin[...]) + blin[...] * (s / den))


def _tc_gru1_body(a0, a1, b0, b1, hv, wet, bet, wih, whh, bih, bhh,
                  wa, wb, bpe, ha_o, hb_o, s1_o, s2_o, h_o):
    ctx = _ctx_from_panels(a0, a1, b0, b1, wet, bet)
    h = jax.nn.relu(_gru(ctx, hv[...], wih[...], whh[...], bih[...], bhh[...]))
    h_o[...] = h
    ha_o[...] = h[:, :PA]
    hb_o[...] = jnp.concatenate(
        [h[:, PA:], jnp.zeros((h.shape[0], PB - (D - PA)), F32)], axis=1)
    s1_o[...] = _mm(h, wa[...]) + bpe[...]
    s2_o[...] = _mm(h, wb[...])


def _tc_gru1(ca, cb, hv, wet, bet, gru_p, wa, wb, bpe):
    blk = 2000
    grid = (N // blk,)
    full = lambda s: pl.BlockSpec(s, lambda i: tuple(0 for _ in s))
    return pl.pallas_call(
        _tc_gru1_body,
        grid=grid,
        in_specs=[
            pl.BlockSpec((blk, PA), lambda i: (i, 0)),
            pl.BlockSpec((blk, PA), lambda i: (i, 0)),
            pl.BlockSpec((blk, PB), lambda i: (i, 0)),
            pl.BlockSpec((blk, PB), lambda i: (i, 0)),
            pl.BlockSpec((blk, D), lambda i: (i, 0)),
            full((D, D)), full((D,)),
            full((D, 3 * D)), full((D, 3 * D)), full((3 * D,)), full((3 * D,)),
            full((D, 1)), full((D, 1)), full((1,)),
        ],
        out_specs=[
            pl.BlockSpec((blk, PA), lambda i: (i, 0)),
            pl.BlockSpec((blk, PB), lambda i: (i, 0)),
            pl.BlockSpec((blk, 1), lambda i: (i, 0)),
            pl.BlockSpec((blk, 1), lambda i: (i, 0)),
            pl.BlockSpec((blk, D), lambda i: (i, 0)),
        ],
        out_shape=[
            jax.ShapeDtypeStruct((N, PA), F32),
            jax.ShapeDtypeStruct((N, PB), F32),
            jax.ShapeDtypeStruct((N, 1), F32),
            jax.ShapeDtypeStruct((N, 1), F32),
            jax.ShapeDtypeStruct((N, D), F32),
        ],
    )(ca[0], ca[1], cb[0], cb[1], hv, wet, bet,
      gru_p['W_ih'], gru_p['W_hh'], gru_p['b_ih'], gru_p['b_hh'],
      wa, wb, bpe)


def _tc_gru2_body(a0, a1, b0, b1, hv, wpn, bpn, wih, whh, bih, bhh, h_o):
    ctx = _ctx_from_panels(a0, a1, b0, b1, wpn, bpn)
    h_o[...] = jax.nn.relu(
        _gru(ctx, hv[...], wih[...], whh[...], bih[...], bhh[...]))


def _tc_gru2(ca, cb, hv, wpn, bpn, gru_p):
    blk = 2000
    grid = (N // blk,)
    full = lambda s: pl.BlockSpec(s, lambda i: tuple(0 for _ in s))
    return pl.pallas_call(
        _tc_gru2_body,
        grid=grid,
        in_specs=[
            pl.BlockSpec((blk, PA), lambda i: (i, 0)),
            pl.BlockSpec((blk, PA), lambda i: (i, 0)),
            pl.BlockSpec((blk, PB), lambda i: (i, 0)),
            pl.BlockSpec((blk, PB), lambda i: (i, 0)),
            pl.BlockSpec((blk, D), lambda i: (i, 0)),
            full((D, D)), full((D,)),
            full((D, 3 * D)), full((D, 3 * D)), full((3 * D,)), full((3 * D,)),
        ],
        out_specs=pl.BlockSpec((blk, D), lambda i: (i, 0)),
        out_shape=jax.ShapeDtypeStruct((N, D), F32),
    )(ca[0], ca[1], cb[0], cb[1], hv, wpn, bpn,
      gru_p['W_ih'], gru_p['W_hh'], gru_p['b_ih'], gru_p['b_hh'])


_CH = 2000


def _af_chunk(gid, k):
    gk = gid[:, k * _CH:(k + 1) * _CH]
    giota = lax.broadcasted_iota(jnp.int32, (G, _CH), 0)
    amask = gk == giota
    return amask, amask.astype(F32)


def _tc_readout_body(h_r, gid_r, rd_r, *refs):
    (wz0a, wz0b, bz0, wp0, bp0, g0ih, g0hh, g0bi, g0bh,
     wz1a, wz1b, bz1, wp1, bp1, g1ih, g1hh, g1bi, g1bh,
     pw1, pb1, pg1, pbe1, pw2, pb2, pg2, pbe2, pw3, pb3, out_r, z_s) = refs
    gid = gid_r[...]
    nch = N // _CH

    g_feats = jnp.zeros((G, D), F32)
    for k in range(nch):
        _, af = _af_chunk(gid, k)
        g_feats = g_feats + _mm(af, h_r[k * _CH:(k + 1) * _CH, :])

    for (wza, wzb, bz, wp, bp, gih, ghh, gbi, gbh) in (
            (wz0a, wz0b, bz0, wp0, bp0, g0ih, g0hh, g0bi, g0bh),
            (wz1a, wz1b, bz1, wp1, bp1, g1ih, g1hh, g1bi, g1bh)):
        gr = jax.nn.relu(g_feats)
        zg = _mm(gr, wza[...])
        m = jnp.full((G, 1), -jnp.inf, F32)
        for k in range(nch):
            amask, af = _af_chunk(gid, k)
            hk = h_r[k * _CH:(k + 1) * _CH, :]
            zrow = _lk(
                lax.dot_general(zg, af, (((0,), (0,)), ((), ())),
                                preferred_element_type=F32)
                + lax.dot_general(wzb[...], hk, (((0,), (1,)), ((), ())),
                                  preferred_element_type=F32)
                + bz[...])
            z_s[:, k * _CH:(k + 1) * _CH] = zrow
            zb = jnp.broadcast_to(zrow, (G, _CH))
            m = jnp.maximum(
                m, jnp.max(jnp.where(amask, zb, -jnp.inf), axis=1,
                           keepdims=True))
        m = jnp.where(jnp.isfinite(m), m, 0.0)
        sacc = jnp.zeros((G, 1), F32)
        for k in range(nch):
            _, af = _af_chunk(gid, k)
            mrow = lax.dot_general(m, af, (((0,), (0,)), ((), ())),
                                   preferred_element_type=F32)
            erow = jnp.exp(z_s[:, k * _CH:(k + 1) * _CH] - mrow)
            z_s[:, k * _CH:(k + 1) * _CH] = erow
            sacc = sacc + lax.dot_general(af, erow, (((1,), (1,)), ((), ())),
                                          preferred_element_type=F32)
        gacc = jnp.zeros((G, D), F32)
        for k in range(nch):
            _, af = _af_chunk(gid, k)
            hk = h_r[k * _CH:(k + 1) * _CH, :]
            srow = lax.dot_general(sacc, af, (((0,), (0,)), ((), ())),
                                   preferred_element_type=F32)
            arow = z_s[:, k * _CH:(k + 1) * _CH] / (srow + 1e-16)
            hvp = _mm(hk, wp[...]) + bp[...]
            gacc = gacc + _mm(af * arow, hvp)
        g_repr = _elu(gacc)
        g_feats = _gru(jax.nn.relu(g_repr), g_feats,
                       gih[...], ghh[...], gbi[...], gbh[...])

    x = jnp.concatenate([g_feats, rd_r[...]], axis=1)
    x = jax.nn.relu(_mm(x, pw1[...]) + pb1[...])
    mu = jnp.mean(x, axis=0, keepdims=True)
    va = jnp.mean((x - mu) ** 2, axis=0, keepdims=True)
    x = (x - mu) / jnp.sqrt(va + 1e-5) * pg1[...] + pbe1[...]
    x = jax.nn.relu(_mm(x, pw2[...]) + pb2[...])
    mu = jnp.mean(x, axis=0, keepdims=True)
    va = jnp.mean((x - mu) ** 2, axis=0, keepdims=True)
    x = (x - mu) / jnp.sqrt(va + 1e-5) * pg2[...] + pbe2[...]
    out_r[...] = _mm(x, pw3[...]) + pb3[...]


def _tc_readout(h, gid_row, rdkit, p):
    args = [h, gid_row, rdkit]
    for t in range(2):
        args += [p['ro%d_W_z' % t][:D], p['ro%d_W_z' % t][D:],
                 p['ro%d_b_z' % t], p['ro%d_W_p' % t], p['ro%d_b_p' % t],
                 p['ro%d_gru' % t]['W_ih'], p['ro%d_gru' % t]['W_hh'],
                 p['ro%d_gru' % t]['b_ih'], p['ro%d_gru' % t]['b_hh']]
    args += [p['pd_W1'], p['pd_b1'], p['pd_g1'], p['pd_be1'],
             p['pd_W2'], p['pd_b2'], p['pd_g2'], p['pd_be2'],
             p['pd_W3'], p['pd_b3']]
    return pl.pallas_call(
        _tc_readout_body,
        out_shape=jax.ShapeDtypeStruct((G, 1), F32),
        scratch_shapes=[pltpu.VMEM((1, N), F32)],
    )(*args)


def _zero_rows(z2d, sh):
    def body(k, _):
        pltpu.sync_copy(z2d, sh.at[pl.ds(k * RW, RW)])
        return 0
    lax.fori_loop(0, N // RW, body, 0)
    pltpu.sync_copy(z2d.at[pl.ds(0, N % RW)], sh.at[pl.ds(N - N % RW, N % RW)])


def _per_worker_rows(body):
    w = _wid()

    def outer(i, _):
        row = w + NW * i

        @pl.when(row < ROWS)
        def _():
            body(row)
        return 0
    lax.fori_loop(0, RPW, outer, 0)


def _scale_rows(rows, av, w, ride_e=False):
    i16 = lax.broadcasted_iota(jnp.int32, (16,), 0)
    def body(j, _):
        v = av[pl.ds(j, 16)]
        s = jnp.broadcast_to(v[0], (16,))
        for g in range(w // 16):
            sl = pl.ds(g * 16, 16)
            x = rows[j, sl] * s
            if ride_e and g == (D - PA) // 16:
                x = jnp.where(i16 == (D - PA) % 16, s, x)
            rows[j, sl] = x
        return 0
    lax.fori_loop(0, RW, body, 0)


def _ctx_copy_out(ctx_sh, out, cid, sid):
    rpt = N // NS
    pltpu.sync_copy(ctx_sh.at[pl.ds(sid * rpt, rpt)],
                    out.at[cid, pl.ds(sid * rpt, rpt)])


@functools.partial(
    pl.kernel, mesh=_MESH, compiler_params=_SC_PARAMS,
    out_type=[jax.ShapeDtypeStruct((E, DP), F32),
              jax.ShapeDtypeStruct((ROWS, RW), F32)],
    scratch_types=[pltpu.VMEM((RW,), jnp.int32),
                   pltpu.VMEM((RW,), jnp.int32),
                   pltpu.VMEM((RW, DP), F32),
                   pltpu.VMEM((N,), F32),
                   pltpu.VMEM((RW,), F32),
                   pltpu.SemaphoreType.DMA],
)
def _sc_gather1(u_h, q_h, src2d, dst2d, u_out, qd_out,
                sidx, didx, rows, qtab, qd, sem):
    pltpu.sync_copy(q_h, qtab)

    def body(row):
        pltpu.sync_copy(src2d.at[row], sidx)
        pltpu.async_copy(u_h.at[sidx], rows, sem).wait()
        pltpu.sync_copy(rows, u_out.at[pl.ds(row * RW, RW)])
        pltpu.sync_copy(dst2d.at[row], didx)
        for g in range(RW // 16):
            qd[pl.ds(g * 16, 16)] = plsc.load_gather(
                qtab, [didx[pl.ds(g * 16, 16)]])
        pltpu.sync_copy(qd, qd_out.at[row])
    _per_worker_rows(body)


def _make_scatter_rows(w):
    @functools.partial(
        pl.kernel, mesh=_MESH, compiler_params=_SC_PARAMS,
        out_type=jax.ShapeDtypeStruct((NC, N, w), F32),
        scratch_types=[pltpu.VMEM((RW,), jnp.int32),
                       pltpu.VMEM((RW, w), F32),
                       pltpu.VMEM_SHARED((N, w), F32)],
    )
    def _k(p_h, dst2d, z2d, ctx_out, didx, rows, ctx_sh):
        cid = lax.axis_index("c")
        sid = lax.axis_index("s")

        @pl.when(sid == 0)
        def _():
            _zero_rows(z2d, ctx_sh)
        plsc.subcore_barrier()

        def body(row):
            pltpu.sync_copy(dst2d.at[row], didx)
            pltpu.sync_copy(p_h.at[pl.ds(row * RW, RW)], rows)
            pltpu.sync_copy(rows, ctx_sh.at[didx], add=True)
        _per_worker_rows(body)
        plsc.subcore_barrier()
        _ctx_copy_out(ctx_sh, ctx_out, cid, sid)
    return _k


_SC_SCATTER = {w: _make_scatter_rows(w) for w in (PA, PB)}


@functools.partial(
    pl.kernel, mesh=_MESH, compiler_params=_SC_PARAMS,
    out_type=jax.ShapeDtypeStruct((ROWS, RW), F32),
    scratch_types=[pltpu.VMEM((RW,), jnp.int32),
                   pltpu.VMEM((RW,), jnp.int32),
                   pltpu.VMEM((RW,), F32),
                   pltpu.VMEM((N,), F32),
                   pltpu.VMEM((N,), F32)],
)
def _sc_edge_logits(s1_h, s2_h, src2d, dst2d, e_out,
                    didx, sidx, ev, t1, t2):
    pltpu.sync_copy(s1_h, t1)
    pltpu.sync_copy(s2_h, t2)

    def body(row):
        pltpu.sync_copy(dst2d.at[row], didx)
        pltpu.sync_copy(src2d.at[row], sidx)
        for g in range(RW // 16):
            sl = pl.ds(g * 16, 16)
            x = (plsc.load_gather(t1, [didx[sl]])
                 + plsc.load_gather(t2, [sidx[sl]]))
            ev[sl] = jnp.exp(jnp.where(x > 0, x, 0.01 * x))
        pltpu.sync_copy(ev, e_out.at[row])
    _per_worker_rows(body)


def _make_gather_scale_scatter(w, ride_e):
    @functools.partial(
        pl.kernel, mesh=_MESH, compiler_params=_SC_PARAMS,
        out_type=jax.ShapeDtypeStruct((NC, N, w), F32),
        scratch_types=[pltpu.VMEM((RW,), jnp.int32),
                       pltpu.VMEM((RW,), jnp.int32),
                       pltpu.VMEM((RW + 16,), F32),
                       pltpu.VMEM((RW, w), F32),
                       pltpu.VMEM_SHARED((N, w), F32),
                       pltpu.SemaphoreType.DMA],
    )
    def _k(hvp_h, a2d, src2d, dst2d, z2d, ctx_out,
           didx, sidx, av, rows, ctx_sh, sem):
        cid = lax.axis_index("c")
        sid = lax.axis_index("s")

        @pl.when(sid == 0)
        def _():
            _zero_rows(z2d, ctx_sh)
        plsc.subcore_barrier()

        def body(row):
            pltpu.sync_copy(dst2d.at[row], didx)
            pltpu.sync_copy(src2d.at[row], sidx)
            pltpu.sync_copy(a2d.at[row], av.at[pl.ds(0, RW)])
            pltpu.async_copy(hvp_h.at[sidx], rows, sem).wait()
            _scale_rows(rows, av, w, ride_e)
            pltpu.sync_copy(rows, ctx_sh.at[didx], add=True)
        _per_worker_rows(body)
        plsc.subcore_barrier()
        _ctx_copy_out(ctx_sh, ctx_out, cid, sid)
    return _k


_SC_GATHER_SCALE_SCATTER = {
    w: _make_gather_scale_scatter(w, w == PB) for w in (PA, PB)}


def kernel(node_feats, edge_feats, rdkitEF, edge_index, node_graph_ids, params):
    p = params
    src = edge_index[0].astype(jnp.int32)
    dst = edge_index[1].astype(jnp.int32)
    src2d = src.reshape(ROWS, RW)
    dst2d = dst.reshape(ROWS, RW)
    za = jnp.zeros((RW, PA), F32)
    zb = jnp.zeros((RW, PB), F32)
    gid_row = node_graph_ids.astype(jnp.int32).reshape(1, N)

    hv, u, q = _tc_node_pre(
        node_feats, p['gc_W_pn'], p['gc_b_pn'],
        p['gc_W_pe1'][:DN], p['gc_b_pe1'],
        p['gc_W_pe2'][:D], p['gc_b_pe2'])
    u_g, qd2d = _sc_gather1(u, q.reshape(N), src2d, dst2d)
    pa, pb = _tc_edge(u_g, edge_feats, qd2d.reshape(E, 1),
                      p['gc_W_pe1'][DN:], p['gc_W_pe2'][D:])
    ctxa = _SC_SCATTER[PA](pa, dst2d, za)
    ctxb = _SC_SCATTER[PB](pb, dst2d, zb)
    ha, hb, s1, s2, h = _tc_gru1(
        ctxa, ctxb, hv, p['gc_W_et'], p['gc_b_et'], p['gc_gru'],
        p['gl_W_pe'][:D], p['gl_W_pe'][D:], p['gl_b_pe'])

    e2_2d = _sc_edge_logits(s1.reshape(N), s2.reshape(N), src2d, dst2d)
    ctx2a = _SC_GATHER_SCALE_SCATTER[PA](ha, e2_2d, src2d, dst2d, za)
    ctx2b = _SC_GATHER_SCALE_SCATTER[PB](hb, e2_2d, src2d, dst2d, zb)
    h2 = _tc_gru2(ctx2a, ctx2b, h, p['gl_W_pn'], p['gl_b_pn'], p['gl_gru'])

    return _tc_readout(h2, gid_row, rdkitEF, p)
